# baseline (speedup 1.0000x reference)
import functools

import jax
import jax.numpy as jnp
from jax.experimental import pallas as pl
from jax.experimental.pallas import tpu as pltpu

K = 20
A = 21
U = 64
L = 512
PDIM = L - K + 1
EPS = 1e-06
BB = 8
BT = 8
AP = 24
GRP = 5


def _r_kernel(p_ref, q_ref, r_ref, rt_ref):
    p = p_ref[...]
    m = jnp.max(p, axis=1, keepdims=True)
    e = jnp.exp(p - m)
    prob = e / jnp.sum(e, axis=1, keepdims=True)
    r = jnp.log(jnp.maximum(prob / q_ref[...], EPS))
    r_ref[...] = r
    rp = jnp.pad(r, ((0, 0), (0, AP - A), (0, 0))).reshape(K * AP, U)
    for g in range(K // GRP):
        rt_ref[g] = rp[g * GRP * AP:(g + 1) * GRP * AP, :].T.astype(
            jnp.bfloat16)


def _t_kernel(x_ref, xt_ref):
    for m in range(BT):
        xm = jnp.pad(x_ref[m], ((0, 0), (0, AP - A)))
        xt_ref[m] = xm.T.astype(jnp.bfloat16)


def _conv_kernel(x_ref, rt_ref, z_ref, s_ref):
    for m in range(BB):
        xmt = x_ref[m]
        zt = jnp.zeros((U, L), dtype=jnp.float32)
        for g in range(K // GRP):
            xcg = jnp.concatenate(
                [xmt if k == 0 else jnp.roll(xmt, -k, axis=1)
                 for k in range(g * GRP, (g + 1) * GRP)],
                axis=0)
            zt = zt + jax.lax.dot_general(
                rt_ref[g], xcg, (((1,), (0,)), ((), ())),
                preferred_element_type=jnp.float32)
        zv = zt.T[:PDIM]
        z_ref[m] = zv
        s_ref[m, :] = jnp.max(zv, axis=0)


@functools.partial(jax.jit, static_argnums=())
def kernel(X, P_logit, Q):
    T, N, F, L_, A_ = X.shape
    B = T * N * F
    Xp = X.reshape(B, L_, A_)

    R, Rt = pl.pallas_call(
        _r_kernel,
        out_shape=(
            jax.ShapeDtypeStruct((K, A, U), jnp.float32),
            jax.ShapeDtypeStruct((K // GRP, U, GRP * AP), jnp.bfloat16),
        ),
    )(P_logit, Q.reshape(1, A, 1))

    Xt = pl.pallas_call(
        _t_kernel,
        grid=(B // BT,),
        in_specs=[pl.BlockSpec((BT, L, A), lambda i: (i, 0, 0))],
        out_specs=pl.BlockSpec((BT, AP, L), lambda i: (i, 0, 0)),
        out_shape=jax.ShapeDtypeStruct((B, AP, L), jnp.bfloat16),
        compiler_params=pltpu.CompilerParams(
            dimension_semantics=("arbitrary",)),
    )(Xp)

    Z, S = pl.pallas_call(
        _conv_kernel,
        grid=(B // BB,),
        in_specs=[
            pl.BlockSpec((BB, AP, L), lambda i: (i, 0, 0)),
            pl.BlockSpec((K // GRP, U, GRP * AP), lambda i: (0, 0, 0)),
        ],
        out_specs=[
            pl.BlockSpec((BB, PDIM, U), lambda i: (i, 0, 0)),
            pl.BlockSpec((BB, U), lambda i: (i, 0)),
        ],
        out_shape=(
            jax.ShapeDtypeStruct((B, PDIM, U), jnp.float32),
            jax.ShapeDtypeStruct((B, U), jnp.float32),
        ),
        compiler_params=pltpu.CompilerParams(
            dimension_semantics=("arbitrary",)),
    )(Xt, Rt)

    return (R, S.reshape(T, N, F, U), Z.reshape(T, N, F, PDIM, U))


# speedup vs baseline: 1.7458x; 1.7458x over previous
import functools

import jax
import jax.numpy as jnp
from jax.experimental import pallas as pl
from jax.experimental.pallas import tpu as pltpu

K = 20
A = 21
U = 64
L = 512
PDIM = L - K + 1
EPS = 1e-06
AP = 24
GRP = 5
LB = 4
NG = 4


def _r_kernel(p_ref, q_ref, r_ref, rt_ref):
    p = p_ref[...]
    m = jnp.max(p, axis=1, keepdims=True)
    e = jnp.exp(p - m)
    prob = e / jnp.sum(e, axis=1, keepdims=True)
    r = jnp.log(jnp.maximum(prob / q_ref[...], EPS))
    r_ref[...] = r
    rp = jnp.pad(r, ((0, 0), (0, AP - A), (0, 0))).reshape(K * AP, U)
    for g in range(K // GRP):
        rt_ref[g] = rp[g * GRP * AP:(g + 1) * GRP * AP, :].T.astype(
            jnp.bfloat16)


def _conv_kernel(x_ref, rt_ref, z_ref, s_ref):
    for n in range(NG):
        xmt = jnp.concatenate(
            [x_ref[0, 0, :, lb * NG + n, :] for lb in range(LB)],
            axis=1)
        xmt = jnp.pad(xmt, ((0, AP - A), (0, 0))).astype(jnp.bfloat16)
        zt = jnp.zeros((U, L), dtype=jnp.float32)
        for g in range(K // GRP):
            xcg = jnp.concatenate(
                [xmt if k == 0 else jnp.roll(xmt, -k, axis=1)
                 for k in range(g * GRP, (g + 1) * GRP)],
                axis=0)
            zt = zt + jax.lax.dot_general(
                rt_ref[g], xcg, (((1,), (0,)), ((), ())),
                preferred_element_type=jnp.float32)
        zv = zt[:, :PDIM]
        z_ref[0, n, 0] = zv
        s_ref[0, 0, n, :] = jnp.max(zv, axis=1)


@functools.partial(jax.jit, static_argnums=())
def kernel(X, P_logit, Q):
    T, N, F, L_, A_ = X.shape
    Xv = (X.transpose(0, 2, 4, 3, 1)
           .reshape(T, F, A_, LB, 128, N)
           .transpose(0, 1, 2, 3, 5, 4)
           .reshape(T, F, A_, LB * N, 128))

    R, Rt = pl.pallas_call(
        _r_kernel,
        out_shape=(
            jax.ShapeDtypeStruct((K, A, U), jnp.float32),
            jax.ShapeDtypeStruct((K // GRP, U, GRP * AP), jnp.bfloat16),
        ),
    )(P_logit, Q.reshape(1, A, 1))

    Zc, Sc = pl.pallas_call(
        _conv_kernel,
        grid=(T, F),
        in_specs=[
            pl.BlockSpec((1, 1, A, NG * LB, 128),
                         lambda t, f: (t, f, 0, 0, 0)),
            pl.BlockSpec((K // GRP, U, GRP * AP), lambda t, f: (0, 0, 0)),
        ],
        out_specs=[
            pl.BlockSpec((1, N, 1, U, PDIM), lambda t, f: (t, 0, f, 0, 0)),
            pl.BlockSpec((1, 1, N, U), lambda t, f: (t, f, 0, 0)),
        ],
        out_shape=(
            jax.ShapeDtypeStruct((T, N, F, U, PDIM), jnp.float32),
            jax.ShapeDtypeStruct((T, F, N, U), jnp.float32),
        ),
        compiler_params=pltpu.CompilerParams(
            dimension_semantics=("arbitrary", "arbitrary")),
    )(Xv, Rt)

    Z = Zc.transpose(0, 1, 2, 4, 3)
    S = Sc.transpose(0, 2, 1, 3)
    return (R, S, Z)


# speedup vs baseline: 2.3906x; 1.3693x over previous
import functools

import jax
import jax.numpy as jnp
from jax.experimental import pallas as pl
from jax.experimental.pallas import tpu as pltpu

K = 20
A = 21
U = 64
L = 512
PDIM = L - K + 1
EPS = 1e-06
AP = 32
GRP = 4
LB = 4
NG = 4


def _r_kernel(p_ref, q_ref, r_ref, rt_ref):
    p = p_ref[...]
    m = jnp.max(p, axis=1, keepdims=True)
    e = jnp.exp(p - m)
    prob = e / jnp.sum(e, axis=1, keepdims=True)
    r = jnp.log(jnp.maximum(prob / q_ref[...], EPS))
    r_ref[...] = r
    rp = jnp.pad(r, ((0, 0), (0, AP - A), (0, 0))).reshape(K * AP, U)
    for g in range(K // GRP):
        rt_ref[g] = rp[g * GRP * AP:(g + 1) * GRP * AP, :].T.astype(
            jnp.bfloat16)


def _conv_kernel(x_ref, rt_ref, z_ref, s_ref):
    xmt = jnp.concatenate(
        [x_ref[0, 0, :, lb * NG + n, :]
         for n in range(NG) for lb in range(LB)],
        axis=1)
    xmt = jnp.pad(xmt, ((0, AP - A), (0, 0))).astype(jnp.bfloat16)
    acc = [jnp.zeros((U, NG * L), dtype=jnp.float32),
           jnp.zeros((U, NG * L), dtype=jnp.float32)]
    for g in range(K // GRP):
        xcg = jnp.concatenate(
            [xmt if k == 0 else jnp.roll(xmt, -k, axis=1)
             for k in range(g * GRP, (g + 1) * GRP)],
            axis=0)
        acc[g % 2] = acc[g % 2] + jax.lax.dot_general(
            rt_ref[g], xcg, (((1,), (0,)), ((), ())),
            preferred_element_type=jnp.float32)
    zt = acc[0] + acc[1]
    for n in range(NG):
        zv = zt[:, n * L:n * L + PDIM]
        z_ref[0, n, 0] = zv
        s_ref[0, 0, n, :] = jnp.max(zv, axis=1)


@functools.partial(jax.jit, static_argnums=())
def kernel(X, P_logit, Q):
    T, N, F, L_, A_ = X.shape
    Xv = (X.transpose(0, 2, 4, 3, 1)
           .reshape(T, F, A_, LB, 128, N)
           .transpose(0, 1, 2, 3, 5, 4)
           .reshape(T, F, A_, LB * N, 128))

    R, Rt = pl.pallas_call(
        _r_kernel,
        out_shape=(
            jax.ShapeDtypeStruct((K, A, U), jnp.float32),
            jax.ShapeDtypeStruct((K // GRP, U, GRP * AP), jnp.bfloat16),
        ),
    )(P_logit, Q.reshape(1, A, 1))

    Zc, Sc = pl.pallas_call(
        _conv_kernel,
        grid=(T, F),
        in_specs=[
            pl.BlockSpec((1, 1, A, NG * LB, 128),
                         lambda t, f: (t, f, 0, 0, 0)),
            pl.BlockSpec((K // GRP, U, GRP * AP), lambda t, f: (0, 0, 0)),
        ],
        out_specs=[
            pl.BlockSpec((1, N, 1, U, PDIM), lambda t, f: (t, 0, f, 0, 0)),
            pl.BlockSpec((1, 1, N, U), lambda t, f: (t, f, 0, 0)),
        ],
        out_shape=(
            jax.ShapeDtypeStruct((T, N, F, U, PDIM), jnp.float32),
            jax.ShapeDtypeStruct((T, F, N, U), jnp.float32),
        ),
        compiler_params=pltpu.CompilerParams(
            dimension_semantics=("arbitrary", "arbitrary")),
    )(Xv, Rt)

    Z = Zc.transpose(0, 1, 2, 4, 3)
    S = Sc.transpose(0, 2, 1, 3)
    return (R, S, Z)
